# baseline (speedup 1.0000x reference)
import functools

import jax
import jax.numpy as jnp
from jax import lax
from jax.experimental import pallas as pl
from jax.experimental.pallas import tpu as pltpu
from jax.experimental.pallas import tpu_sc as plsc

B = 16384
C = 1000
NC = 2
NS = 16
NW = NC * NS
BPW = B // NW
BT = 128
NK = BPW // BT
CC = 200
NM = C // CC
NCH = NK * NM

_mesh = plsc.VectorSubcoreMesh(core_axis_name="c", subcore_axis_name="s")


@functools.partial(
    pl.kernel,
    mesh=_mesh,
    out_type=jax.ShapeDtypeStruct((C, B), jnp.float32),
    compiler_params=pltpu.CompilerParams(
        needs_layout_passes=False, skip_device_barrier=True
    ),
    scratch_types=[
        pltpu.VMEM((BPW,), jnp.int32),
        pltpu.VMEM((CC, BT), jnp.float32),
        pltpu.VMEM((CC, BT), jnp.float32),
        pltpu.SemaphoreType.DMA,
        pltpu.SemaphoreType.DMA,
    ],
)
def _onehot_sc(in_hbm, out_hbm, idx_v, buf0, buf1, sem0, sem1):
    wid = lax.axis_index("s") * NC + lax.axis_index("c")
    base_col = wid * BPW

    pltpu.sync_copy(in_hbm.at[pl.ds(base_col, BPW)], idx_v)

    lanes = lax.iota(jnp.int32, 16)
    ones = jnp.full((16,), 1.0, jnp.float32)
    zeros = jnp.zeros((16,), jnp.float32)

    def _zero(r, carry):
        for j in range(BT // 16):
            buf0[r, pl.ds(j * 16, 16)] = zeros
            buf1[r, pl.ds(j * 16, 16)] = zeros
        return carry

    lax.fori_loop(0, CC, _zero, 0)

    bufs = (buf0, buf1)
    sems = (sem0, sem1)

    def dst(ci):
        k = ci // NM
        m = ci % NM
        c0 = pl.multiple_of(m * CC, 8)
        b0 = pl.multiple_of(base_col + k * BT, 128)
        return out_hbm.at[pl.ds(c0, CC), pl.ds(b0, BT)]

    def scatter(buf, ci, val):
        k = ci // NM
        c0 = (ci % NM) * CC
        for v in range(8):
            cols = idx_v[pl.ds(k * BT + v * 16, 16)]
            local_c = cols - c0
            mask = (local_c >= 0) & (local_c < CC)
            local_c = jnp.where(mask, local_c, 0)
            local_b = lanes + (v * 16)
            plsc.store_scatter(buf, [local_c, local_b], val, mask=mask)

    for b in range(2):
        scatter(bufs[b], b, ones)
        pltpu.async_copy(bufs[b], dst(b), sems[b])

    def _body(c2, carry):
        for b in range(2):
            ci = c2 * 2 + b
            buf, sem = bufs[b], sems[b]
            pltpu.make_async_copy(buf, dst(ci), sem).wait()
            scatter(buf, ci - 2, zeros)
            scatter(buf, ci, ones)
            pltpu.async_copy(buf, dst(ci), sem)
        return carry

    lax.fori_loop(1, NCH // 2, _body, 0)

    for b in range(2):
        pltpu.make_async_copy(bufs[b], dst(NCH - 2 + b), sems[b]).wait()


def kernel(inputs):
    out_t = _onehot_sc(inputs.astype(jnp.int32))
    return out_t.T


# speedup vs baseline: 1.0095x; 1.0095x over previous
import functools

import jax
import jax.numpy as jnp
from jax import lax
from jax.experimental import pallas as pl
from jax.experimental.pallas import tpu as pltpu
from jax.experimental.pallas import tpu_sc as plsc

B = 16384
C = 1000
NC = 2
NS = 16
NW = NC * NS
BPW = B // NW
BT = 128
NK = BPW // BT
CC = 200
NM = C // CC
NCH = NK * NM

_mesh = plsc.VectorSubcoreMesh(core_axis_name="c", subcore_axis_name="s")


@functools.partial(
    pl.kernel,
    mesh=_mesh,
    out_type=jax.ShapeDtypeStruct((C, B), jnp.float32),
    compiler_params=pltpu.CompilerParams(needs_layout_passes=False),
    scratch_types=[
        pltpu.VMEM((BPW,), jnp.int32),
        pltpu.VMEM((CC, BT), jnp.float32),
        pltpu.VMEM((CC, BT), jnp.float32),
        pltpu.SemaphoreType.DMA,
        pltpu.SemaphoreType.DMA,
    ],
)
def _onehot_sc(in_hbm, out_hbm, idx_v, buf0, buf1, sem0, sem1):
    wid = lax.axis_index("s") * NC + lax.axis_index("c")
    base_col = wid * BPW

    pltpu.sync_copy(in_hbm.at[pl.ds(base_col, BPW)], idx_v)

    lanes = lax.iota(jnp.int32, 16)
    ones = jnp.full((16,), 1.0, jnp.float32)
    zeros = jnp.zeros((16,), jnp.float32)

    def _zero(r, carry):
        for j in range(BT // 16):
            buf0[r, pl.ds(j * 16, 16)] = zeros
            buf1[r, pl.ds(j * 16, 16)] = zeros
        return carry

    lax.fori_loop(0, CC, _zero, 0)

    bufs = (buf0, buf1)
    sems = (sem0, sem1)

    def dst(ci):
        k = ci // NM
        m = ci % NM
        c0 = pl.multiple_of(m * CC, 8)
        b0 = pl.multiple_of(base_col + k * BT, 128)
        return out_hbm.at[pl.ds(c0, CC), pl.ds(b0, BT)]

    def scatter(buf, ci, val):
        k = ci // NM
        c0 = (ci % NM) * CC
        for v in range(8):
            cols = idx_v[pl.ds(k * BT + v * 16, 16)]
            local_c = cols - c0
            mask = (local_c >= 0) & (local_c < CC)
            local_c = jnp.where(mask, local_c, 0)
            local_b = lanes + (v * 16)
            plsc.store_scatter(buf, [local_c, local_b], val, mask=mask)

    def _body(c2, carry):
        for b in range(2):
            ci = c2 * 2 + b
            buf, sem = bufs[b], sems[b]

            @pl.when(c2 > 0)
            def _():
                pltpu.make_async_copy(buf, dst(ci), sem).wait()
                scatter(buf, ci - 2, zeros)

            scatter(buf, ci, ones)
            pltpu.async_copy(buf, dst(ci), sem)
        return carry

    lax.fori_loop(0, NCH // 2, _body, 0)

    for b in range(2):
        pltpu.make_async_copy(bufs[b], dst(NCH - 2 + b), sems[b]).wait()


def kernel(inputs):
    out_t = _onehot_sc(inputs.astype(jnp.int32))
    return out_t.T


# speedup vs baseline: 1.0178x; 1.0082x over previous
import functools

import jax
import jax.numpy as jnp
from jax import lax
from jax.experimental import pallas as pl
from jax.experimental.pallas import tpu as pltpu
from jax.experimental.pallas import tpu_sc as plsc

B = 16384
C = 1000
NC = 2
NS = 16
NW = NC * NS
BPW = B // NW
BT = 128
NK = BPW // BT
CC = 200
NM = C // CC
NCH = NK * NM

_mesh = plsc.VectorSubcoreMesh(core_axis_name="c", subcore_axis_name="s")


@functools.partial(
    pl.kernel,
    mesh=_mesh,
    out_type=jax.ShapeDtypeStruct((C, B), jnp.float32),
    compiler_params=pltpu.CompilerParams(needs_layout_passes=False),
    scratch_types=[
        pltpu.VMEM((BPW,), jnp.int32),
        pltpu.VMEM((CC, BT), jnp.float32),
        pltpu.VMEM((CC, BT), jnp.float32),
        pltpu.SemaphoreType.DMA,
        pltpu.SemaphoreType.DMA,
    ],
)
def _onehot_sc(in_hbm, out_hbm, idx_v, buf0, buf1, sem0, sem1):
    wid = lax.axis_index("s") * NC + lax.axis_index("c")
    base_col = wid * BPW

    idx_cp = pltpu.async_copy(in_hbm.at[pl.ds(base_col, BPW)], idx_v, sem0)

    lanes = lax.iota(jnp.int32, 16)
    ones = jnp.full((16,), 1.0, jnp.float32)
    zeros = jnp.zeros((16,), jnp.float32)

    def _zero(r, carry):
        for j in range(BT // 16):
            buf0[r, pl.ds(j * 16, 16)] = zeros
            buf1[r, pl.ds(j * 16, 16)] = zeros
        return carry

    lax.fori_loop(0, CC, _zero, 0)
    idx_cp.wait()

    bufs = (buf0, buf1)
    sems = (sem0, sem1)

    def dst(ci):
        k = ci // NM
        m = ci % NM
        c0 = pl.multiple_of(m * CC, 8)
        b0 = pl.multiple_of(base_col + k * BT, 128)
        return out_hbm.at[pl.ds(c0, CC), pl.ds(b0, BT)]

    def scatter(buf, ci, val):
        k = ci // NM
        c0 = (ci % NM) * CC
        for v in range(8):
            cols = idx_v[pl.ds(k * BT + v * 16, 16)]
            local_c = cols - c0
            mask = (local_c >= 0) & (local_c < CC)
            local_c = jnp.where(mask, local_c, 0)
            local_b = lanes + (v * 16)
            plsc.store_scatter(buf, [local_c, local_b], val, mask=mask)

    def _body(c2, carry):
        for b in range(2):
            ci = c2 * 2 + b
            buf, sem = bufs[b], sems[b]

            @pl.when(c2 > 0)
            def _():
                pltpu.make_async_copy(buf, dst(ci), sem).wait()
                scatter(buf, ci - 2, zeros)

            scatter(buf, ci, ones)
            pltpu.async_copy(buf, dst(ci), sem)
        return carry

    lax.fori_loop(0, NCH // 2, _body, 0)

    for b in range(2):
        pltpu.make_async_copy(bufs[b], dst(NCH - 2 + b), sems[b]).wait()


def kernel(inputs):
    out_t = _onehot_sc(inputs.astype(jnp.int32))
    return out_t.T
